# baseline (device time: 105297 ns/iter reference)
import jax
import jax.numpy as jnp
from jax import lax
from jax.experimental import pallas as pl
from jax.experimental.pallas import tpu as pltpu

W = 16
M_CH = 256
K_SH = 256
N = 2048
HALF = N // 2
SUBS = 4
COLW = HALF // SUBS
WIRE_DTYPE = jnp.int16
WIRE_SCALE = 2048.0

FLOWS = [(d, d * HALF + k * COLW) for k in range(SUBS) for d in (0, 1)]
NF = len(FLOWS)

_DID_MESH = getattr(pl, "DeviceIdType", getattr(pltpu, "DeviceIdType", None)).MESH
_sem_signal = getattr(pl, "semaphore_signal", None) or pltpu.semaphore_signal
_sem_wait = getattr(pl, "semaphore_wait", None) or pltpu.semaphore_wait
_CompilerParams = getattr(pltpu, "CompilerParams", None) or pltpu.TPUCompilerParams


def kernel(x, w_mat):
    def body(x_ref, w_ref, out_ref, *scr):
        xs_hi, xs_lo, w_hi, w_lo, amax_ref = scr[:5]
        comms = scr[5:5 + NF]
        sbufs = scr[5 + NF:5 + 2 * NF]
        ssems = scr[5 + 2 * NF:5 + 3 * NF]
        rsems = scr[5 + 3 * NF:5 + 4 * NF]
        credits = scr[5 + 4 * NF:5 + 5 * NF]
        assem, arsem = scr[5 + 5 * NF:]

        e = lax.axis_index("i")
        right = lax.rem(e + 1, W)
        left = lax.rem(e - 1 + W, W)
        nbr_of = (right, left)
        src_of = (left, right)

        barrier = pltpu.get_barrier_semaphore()
        for nbr in (left, right):
            _sem_signal(barrier, 1, device_id=(nbr,), device_id_type=_DID_MESH)
        _sem_wait(barrier, 2)

        wsc = w_ref[...] * WIRE_SCALE
        whi = wsc.astype(jnp.bfloat16)
        w_hi[...] = whi
        w_lo[...] = (wsc - whi.astype(jnp.float32)).astype(jnp.bfloat16)
        xv = x_ref[...]
        xhi = xv.astype(jnp.bfloat16)
        xs_hi[...] = xhi
        xs_lo[...] = (xv - xhi.astype(jnp.float32)).astype(jnp.bfloat16)

        def dir_gemm(c, d):
            xh = xs_hi[pl.ds(c * M_CH, M_CH), :]
            xl = xs_lo[pl.ds(c * M_CH, M_CH), :]
            wh = w_hi[:, d * HALF:(d + 1) * HALF]
            wl = w_lo[:, d * HALF:(d + 1) * HALF]
            def dot(a, b):
                return lax.dot_general(
                    a, b, dimension_numbers=(((1,), (0,)), ((), ())),
                    preferred_element_type=jnp.float32)
            return dot(xh, wh) + (dot(xh, wl) + dot(xl, wh))

        def chunk_idx(s, d):
            return lax.rem(e - 1 - s + 2 * W, W) if d == 0 \
                else lax.rem(e + 1 + s, W)

        rd = [[] for _ in range(NF)]
        for s in range(W - 1):
            g = (dir_gemm(chunk_idx(s, 0), 0), dir_gemm(chunk_idx(s, 1), 1))
            for fi, (d, c0) in enumerate(FLOWS):
                rel = c0 - d * HALF
                part = g[d][:, rel:rel + COLW]
                if s > 0:
                    rd[fi][s - 1].wait_recv()
                    part = part + comms[fi][(s - 1) % 2].astype(jnp.float32)
                    if s <= W - 3:
                        _sem_signal(credits[fi], 1, device_id=(src_of[d],),
                                    device_id_type=_DID_MESH)
                if s >= 2:
                    rd[fi][s - 2].wait_send()
                sbufs[fi][s % 2] = jnp.round(part).astype(WIRE_DTYPE)
                if s >= 2:
                    _sem_wait(credits[fi], 1)
                r = pltpu.make_async_remote_copy(
                    src_ref=sbufs[fi].at[s % 2],
                    dst_ref=comms[fi].at[s % 2],
                    send_sem=ssems[fi].at[s % 2],
                    recv_sem=rsems[fi].at[s % 2],
                    device_id=(nbr_of[d],),
                    device_id_type=_DID_MESH,
                )
                r.start()
                rd[fi].append(r)

        gf = (dir_gemm(e, 0), dir_gemm(e, 1))
        ys = []
        for fi, (d, c0) in enumerate(FLOWS):
            rel = c0 - d * HALF
            rd[fi][W - 2].wait_recv()
            acc = comms[fi][(W - 2) % 2].astype(jnp.float32) \
                + gf[d][:, rel:rel + COLW]
            ys.append(acc * (1.0 / WIRE_SCALE))
            rd[fi][W - 3].wait_send()
            rd[fi][W - 2].wait_send()

        amax_l = jnp.max(jnp.abs(ys[0]))
        for yf in ys[1:]:
            amax_l = jnp.maximum(amax_l, jnp.max(jnp.abs(yf)))
        amax_ref[pl.ds(e, 1), :] = jnp.full((1, 128), amax_l, jnp.float32)
        rdmas = []
        for o in range(1, W):
            t = lax.rem(e + o, W)
            r = pltpu.make_async_remote_copy(
                src_ref=amax_ref.at[pl.ds(e, 1)],
                dst_ref=amax_ref.at[pl.ds(e, 1)],
                send_sem=assem.at[o],
                recv_sem=arsem.at[o],
                device_id=(t,),
                device_id_type=_DID_MESH,
            )
            r.start()
            rdmas.append(r)
        for r in rdmas:
            r.wait_send()
        for r in rdmas:
            r.wait_recv()

        amax_g = jnp.max(amax_ref[...])
        scale = amax_g / 448.0
        for fi, (d, c0) in enumerate(FLOWS):
            q = (ys[fi] / scale).astype(jnp.float8_e4m3fn)
            out_ref[:, c0:c0 + COLW] = q.astype(jnp.float32) * scale

    scratch = [
        pltpu.VMEM((W * M_CH, K_SH), jnp.bfloat16),
        pltpu.VMEM((W * M_CH, K_SH), jnp.bfloat16),
        pltpu.VMEM((K_SH, N), jnp.bfloat16),
        pltpu.VMEM((K_SH, N), jnp.bfloat16),
        pltpu.VMEM((W, 128), jnp.float32),
    ]
    scratch += [pltpu.VMEM((2, M_CH, COLW), WIRE_DTYPE) for _ in range(NF)]
    scratch += [pltpu.VMEM((2, M_CH, COLW), WIRE_DTYPE) for _ in range(NF)]
    scratch += [pltpu.SemaphoreType.DMA((2,)) for _ in range(NF)]
    scratch += [pltpu.SemaphoreType.DMA((2,)) for _ in range(NF)]
    scratch += [pltpu.SemaphoreType.REGULAR for _ in range(NF)]
    scratch += [pltpu.SemaphoreType.DMA((W,)),
                pltpu.SemaphoreType.DMA((W,))]

    return pl.pallas_call(
        body,
        out_shape=jax.ShapeDtypeStruct((M_CH, N), jnp.float32),
        in_specs=[
            pl.BlockSpec(memory_space=pltpu.VMEM),
            pl.BlockSpec(memory_space=pltpu.VMEM),
        ],
        out_specs=pl.BlockSpec(memory_space=pltpu.VMEM),
        scratch_shapes=scratch,
        compiler_params=_CompilerParams(collective_id=0),
    )(x, w_mat)


# device time: 104868 ns/iter; 1.0041x vs baseline; 1.0041x over previous
import jax
import jax.numpy as jnp
from jax import lax
from jax.experimental import pallas as pl
from jax.experimental.pallas import tpu as pltpu

W = 16
M_CH = 256
K_SH = 256
N = 2048
HALF = N // 2
SUBS = 2
COLW = HALF // SUBS
WIRE_DTYPE = jnp.int16
WIRE_SCALE = 2048.0

FLOWS = [(d, d * HALF + k * COLW) for k in range(SUBS) for d in (0, 1)]
NF = len(FLOWS)

_DID_MESH = getattr(pl, "DeviceIdType", getattr(pltpu, "DeviceIdType", None)).MESH
_sem_signal = getattr(pl, "semaphore_signal", None) or pltpu.semaphore_signal
_sem_wait = getattr(pl, "semaphore_wait", None) or pltpu.semaphore_wait
_CompilerParams = getattr(pltpu, "CompilerParams", None) or pltpu.TPUCompilerParams


def kernel(x, w_mat):
    def body(x_ref, w_ref, out_ref, *scr):
        xs_hi, xs_lo, w_hi, w_lo, amax_ref = scr[:5]
        comms = scr[5:5 + NF]
        sbufs = scr[5 + NF:5 + 2 * NF]
        ssems = scr[5 + 2 * NF:5 + 3 * NF]
        rsems = scr[5 + 3 * NF:5 + 4 * NF]
        credits = scr[5 + 4 * NF:5 + 5 * NF]
        assem, arsem = scr[5 + 5 * NF:]

        e = lax.axis_index("i")
        right = lax.rem(e + 1, W)
        left = lax.rem(e - 1 + W, W)
        nbr_of = (right, left)
        src_of = (left, right)

        barrier = pltpu.get_barrier_semaphore()
        for nbr in (left, right):
            _sem_signal(barrier, 1, device_id=(nbr,), device_id_type=_DID_MESH)
        _sem_wait(barrier, 2)

        wsc = w_ref[...] * WIRE_SCALE
        whi = wsc.astype(jnp.bfloat16)
        w_hi[...] = whi
        w_lo[...] = (wsc - whi.astype(jnp.float32)).astype(jnp.bfloat16)
        xv = x_ref[...]
        xhi = xv.astype(jnp.bfloat16)
        xs_hi[...] = xhi
        xs_lo[...] = (xv - xhi.astype(jnp.float32)).astype(jnp.bfloat16)

        def dir_gemm(c, d):
            xh = xs_hi[pl.ds(c * M_CH, M_CH), :]
            xl = xs_lo[pl.ds(c * M_CH, M_CH), :]
            wh = w_hi[:, d * HALF:(d + 1) * HALF]
            wl = w_lo[:, d * HALF:(d + 1) * HALF]
            def dot(a, b):
                return lax.dot_general(
                    a, b, dimension_numbers=(((1,), (0,)), ((), ())),
                    preferred_element_type=jnp.float32)
            return dot(xh, wh) + (dot(xh, wl) + dot(xl, wh))

        def chunk_idx(s, d):
            return lax.rem(e - 1 - s + 2 * W, W) if d == 0 \
                else lax.rem(e + 1 + s, W)

        rd = [[] for _ in range(NF)]
        for s in range(W - 1):
            g = (dir_gemm(chunk_idx(s, 0), 0), dir_gemm(chunk_idx(s, 1), 1))
            for fi, (d, c0) in enumerate(FLOWS):
                rel = c0 - d * HALF
                part = jnp.round(g[d][:, rel:rel + COLW]).astype(WIRE_DTYPE)
                if s > 0:
                    rd[fi][s - 1].wait_recv()
                    part = part + comms[fi][(s - 1) % 2]
                    if s <= W - 3:
                        _sem_signal(credits[fi], 1, device_id=(src_of[d],),
                                    device_id_type=_DID_MESH)
                if s >= 2:
                    rd[fi][s - 2].wait_send()
                sbufs[fi][s % 2] = part
                if s >= 2:
                    _sem_wait(credits[fi], 1)
                r = pltpu.make_async_remote_copy(
                    src_ref=sbufs[fi].at[s % 2],
                    dst_ref=comms[fi].at[s % 2],
                    send_sem=ssems[fi].at[s % 2],
                    recv_sem=rsems[fi].at[s % 2],
                    device_id=(nbr_of[d],),
                    device_id_type=_DID_MESH,
                )
                r.start()
                rd[fi].append(r)

        gf = (dir_gemm(e, 0), dir_gemm(e, 1))
        ys = []
        for fi, (d, c0) in enumerate(FLOWS):
            rel = c0 - d * HALF
            rd[fi][W - 2].wait_recv()
            acc = comms[fi][(W - 2) % 2] \
                + jnp.round(gf[d][:, rel:rel + COLW]).astype(WIRE_DTYPE)
            ys.append(acc.astype(jnp.float32) * (1.0 / WIRE_SCALE))
            rd[fi][W - 3].wait_send()
            rd[fi][W - 2].wait_send()

        amax_l = jnp.max(jnp.abs(ys[0]))
        for yf in ys[1:]:
            amax_l = jnp.maximum(amax_l, jnp.max(jnp.abs(yf)))
        amax_ref[pl.ds(e, 1), :] = jnp.full((1, 128), amax_l, jnp.float32)
        rdmas = []
        for o in range(1, W):
            t = lax.rem(e + o, W)
            r = pltpu.make_async_remote_copy(
                src_ref=amax_ref.at[pl.ds(e, 1)],
                dst_ref=amax_ref.at[pl.ds(e, 1)],
                send_sem=assem.at[o],
                recv_sem=arsem.at[o],
                device_id=(t,),
                device_id_type=_DID_MESH,
            )
            r.start()
            rdmas.append(r)
        for r in rdmas:
            r.wait_send()
        for r in rdmas:
            r.wait_recv()

        amax_g = jnp.max(amax_ref[...])
        scale = amax_g / 448.0
        for fi, (d, c0) in enumerate(FLOWS):
            q = (ys[fi] / scale).astype(jnp.float8_e4m3fn)
            out_ref[:, c0:c0 + COLW] = q.astype(jnp.float32) * scale

    scratch = [
        pltpu.VMEM((W * M_CH, K_SH), jnp.bfloat16),
        pltpu.VMEM((W * M_CH, K_SH), jnp.bfloat16),
        pltpu.VMEM((K_SH, N), jnp.bfloat16),
        pltpu.VMEM((K_SH, N), jnp.bfloat16),
        pltpu.VMEM((W, 128), jnp.float32),
    ]
    scratch += [pltpu.VMEM((2, M_CH, COLW), WIRE_DTYPE) for _ in range(NF)]
    scratch += [pltpu.VMEM((2, M_CH, COLW), WIRE_DTYPE) for _ in range(NF)]
    scratch += [pltpu.SemaphoreType.DMA((2,)) for _ in range(NF)]
    scratch += [pltpu.SemaphoreType.DMA((2,)) for _ in range(NF)]
    scratch += [pltpu.SemaphoreType.REGULAR for _ in range(NF)]
    scratch += [pltpu.SemaphoreType.DMA((W,)),
                pltpu.SemaphoreType.DMA((W,))]

    return pl.pallas_call(
        body,
        out_shape=jax.ShapeDtypeStruct((M_CH, N), jnp.float32),
        in_specs=[
            pl.BlockSpec(memory_space=pltpu.VMEM),
            pl.BlockSpec(memory_space=pltpu.VMEM),
        ],
        out_specs=pl.BlockSpec(memory_space=pltpu.VMEM),
        scratch_shapes=scratch,
        compiler_params=_CompilerParams(collective_id=0),
    )(x, w_mat)


# device time: 103747 ns/iter; 1.0149x vs baseline; 1.0108x over previous
import jax
import jax.numpy as jnp
from jax import lax
from jax.experimental import pallas as pl
from jax.experimental.pallas import tpu as pltpu

W = 16
M_CH = 256
K_SH = 256
N = 2048
HALF = N // 2
SUBS = 2
COLW = HALF // SUBS
WIRE_DTYPE = jnp.int16
WIRE_SCALE = 2048.0

FLOWS = [(d, d * HALF + k * COLW) for k in range(SUBS) for d in (0, 1)]
NF = len(FLOWS)

_DID_MESH = getattr(pl, "DeviceIdType", getattr(pltpu, "DeviceIdType", None)).MESH
_sem_signal = getattr(pl, "semaphore_signal", None) or pltpu.semaphore_signal
_sem_wait = getattr(pl, "semaphore_wait", None) or pltpu.semaphore_wait
_CompilerParams = getattr(pltpu, "CompilerParams", None) or pltpu.TPUCompilerParams


def kernel(x, w_mat):
    def body(x_ref, w_ref, out_ref, *scr):
        w_hi, w_lo, amax_ref = scr[:3]
        comms = scr[3:3 + NF]
        sbufs = scr[3 + NF:3 + 2 * NF]
        ssems = scr[3 + 2 * NF:3 + 3 * NF]
        rsems = scr[3 + 3 * NF:3 + 4 * NF]
        credits = scr[3 + 4 * NF:3 + 5 * NF]
        assem, arsem = scr[3 + 5 * NF:]

        e = lax.axis_index("i")
        right = lax.rem(e + 1, W)
        left = lax.rem(e - 1 + W, W)
        nbr_of = (right, left)
        src_of = (left, right)

        barrier = pltpu.get_barrier_semaphore()
        for nbr in (left, right):
            _sem_signal(barrier, 1, device_id=(nbr,), device_id_type=_DID_MESH)
        _sem_wait(barrier, 2)

        wsc = w_ref[...] * WIRE_SCALE
        whi = wsc.astype(jnp.bfloat16)
        w_hi[...] = whi
        w_lo[...] = (wsc - whi.astype(jnp.float32)).astype(jnp.bfloat16)

        def dir_gemm(c, d):
            xc = x_ref[pl.ds(c * M_CH, M_CH), :]
            xh = xc.astype(jnp.bfloat16)
            xl = (xc - xh.astype(jnp.float32)).astype(jnp.bfloat16)
            wh = w_hi[:, d * HALF:(d + 1) * HALF]
            wl = w_lo[:, d * HALF:(d + 1) * HALF]
            def dot(a, b):
                return lax.dot_general(
                    a, b, dimension_numbers=(((1,), (0,)), ((), ())),
                    preferred_element_type=jnp.float32)
            return dot(xh, wh) + (dot(xh, wl) + dot(xl, wh))

        def chunk_idx(s, d):
            return lax.rem(e - 1 - s + 2 * W, W) if d == 0 \
                else lax.rem(e + 1 + s, W)

        rd = [[] for _ in range(NF)]
        for s in range(W - 1):
            g = (dir_gemm(chunk_idx(s, 0), 0), dir_gemm(chunk_idx(s, 1), 1))
            for fi, (d, c0) in enumerate(FLOWS):
                rel = c0 - d * HALF
                part = jnp.round(g[d][:, rel:rel + COLW]).astype(WIRE_DTYPE)
                if s > 0:
                    rd[fi][s - 1].wait_recv()
                    part = part + comms[fi][(s - 1) % 2]
                    if s <= W - 3:
                        _sem_signal(credits[fi], 1, device_id=(src_of[d],),
                                    device_id_type=_DID_MESH)
                if s >= 2:
                    rd[fi][s - 2].wait_send()
                sbufs[fi][s % 2] = part
                if s >= 2:
                    _sem_wait(credits[fi], 1)
                r = pltpu.make_async_remote_copy(
                    src_ref=sbufs[fi].at[s % 2],
                    dst_ref=comms[fi].at[s % 2],
                    send_sem=ssems[fi].at[s % 2],
                    recv_sem=rsems[fi].at[s % 2],
                    device_id=(nbr_of[d],),
                    device_id_type=_DID_MESH,
                )
                r.start()
                rd[fi].append(r)

        gf = (dir_gemm(e, 0), dir_gemm(e, 1))
        accs = []
        for fi, (d, c0) in enumerate(FLOWS):
            rel = c0 - d * HALF
            rd[fi][W - 2].wait_recv()
            acc = comms[fi][(W - 2) % 2] \
                + jnp.round(gf[d][:, rel:rel + COLW]).astype(WIRE_DTYPE)
            accs.append(acc)
            rd[fi][W - 3].wait_send()
            rd[fi][W - 2].wait_send()

        amax_l = jnp.max(jnp.abs(accs[0].astype(jnp.float32)))
        for af in accs[1:]:
            amax_l = jnp.maximum(amax_l, jnp.max(jnp.abs(af.astype(jnp.float32))))
        amax_ref[pl.ds(e, 1), :] = jnp.full((1, 128), amax_l, jnp.float32)
        rdmas = []
        for o in range(1, W):
            t = lax.rem(e + o, W)
            r = pltpu.make_async_remote_copy(
                src_ref=amax_ref.at[pl.ds(e, 1)],
                dst_ref=amax_ref.at[pl.ds(e, 1)],
                send_sem=assem.at[o],
                recv_sem=arsem.at[o],
                device_id=(t,),
                device_id_type=_DID_MESH,
            )
            r.start()
            rdmas.append(r)
        for r in rdmas:
            r.wait_send()
        for r in rdmas:
            r.wait_recv()

        amax_g_sc = jnp.max(amax_ref[...])
        qmul = 448.0 / amax_g_sc
        deq = amax_g_sc / (448.0 * WIRE_SCALE)
        for fi, (d, c0) in enumerate(FLOWS):
            q = (accs[fi].astype(jnp.float32) * qmul).astype(jnp.float8_e4m3fn)
            out_ref[:, c0:c0 + COLW] = q.astype(jnp.float32) * deq

    scratch = [
        pltpu.VMEM((K_SH, N), jnp.bfloat16),
        pltpu.VMEM((K_SH, N), jnp.bfloat16),
        pltpu.VMEM((W, 128), jnp.float32),
    ]
    scratch += [pltpu.VMEM((2, M_CH, COLW), WIRE_DTYPE) for _ in range(NF)]
    scratch += [pltpu.VMEM((2, M_CH, COLW), WIRE_DTYPE) for _ in range(NF)]
    scratch += [pltpu.SemaphoreType.DMA((2,)) for _ in range(NF)]
    scratch += [pltpu.SemaphoreType.DMA((2,)) for _ in range(NF)]
    scratch += [pltpu.SemaphoreType.REGULAR for _ in range(NF)]
    scratch += [pltpu.SemaphoreType.DMA((W,)),
                pltpu.SemaphoreType.DMA((W,))]

    return pl.pallas_call(
        body,
        out_shape=jax.ShapeDtypeStruct((M_CH, N), jnp.float32),
        in_specs=[
            pl.BlockSpec(memory_space=pltpu.VMEM),
            pl.BlockSpec(memory_space=pltpu.VMEM),
        ],
        out_specs=pl.BlockSpec(memory_space=pltpu.VMEM),
        scratch_shapes=scratch,
        compiler_params=_CompilerParams(collective_id=0),
    )(x, w_mat)
